# baseline (speedup 1.0000x reference)
import functools

import jax
import jax.numpy as jnp
from jax import lax
from jax.experimental import pallas as pl
from jax.experimental.pallas import tpu as pltpu
from jax.experimental.pallas import tpu_sc as plsc

B, T, D, S = 8, 2048, 512, 256
TW = 256
L = 16
NW = 32
SPW = (B * S) // NW


HB = 4


def _tc_logits_body(seq_ref, w_ref, b_ref, expz_ref):
    seq = seq_ref[...].reshape(HB * TW, D)
    z = jnp.dot(seq, w_ref[...], preferred_element_type=jnp.float32)
    z = z.reshape(HB, TW) + b_ref[0]
    mx = jnp.max(z, axis=1, keepdims=True)
    expz_ref[...] = jnp.exp(z - mx).reshape(HB, 1, TW)


def _tc_logits(sequence_tensor, w, b):
    return pl.pallas_call(
        _tc_logits_body,
        grid=(B // HB,),
        in_specs=[
            pl.BlockSpec((HB, TW, D), lambda i: (i, 0, 0)),
            pl.BlockSpec((D, 1), lambda i: (0, 0)),
            pl.BlockSpec((1,), lambda i: (0,)),
        ],
        out_specs=pl.BlockSpec((HB, 1, TW), lambda i: (i, 0, 0)),
        out_shape=jax.ShapeDtypeStruct((B, 1, TW), jnp.float32),
    )(sequence_tensor, w, b)


_sc_mesh = plsc.VectorSubcoreMesh(core_axis_name="c", subcore_axis_name="s")


@functools.partial(
    pl.kernel,
    mesh=_sc_mesh,
    out_type=jax.ShapeDtypeStruct((B * S, TW), jnp.float32),
    scratch_types=[
        pltpu.VMEM((SPW * 2,), jnp.int32),
        pltpu.VMEM((SPW, TW), jnp.float32),
        pltpu.SemaphoreType.DMA,
    ],
)
def _sc_build_mask(spans_hbm, a_hbm, spans_v, a_v, sem):
    wid = lax.axis_index("s") * 2 + lax.axis_index("c")
    r0 = wid * SPW
    pltpu.sync_copy(spans_hbm.at[pl.ds(r0 * 2, SPW * 2)], spans_v)

    iota = lax.iota(jnp.int32, L)
    lanes_j = [iota + j * L for j in range(TW // L)]
    one = jnp.full((L,), 1.0, dtype=jnp.float32)
    zero = jnp.zeros((L,), dtype=jnp.float32)
    GP = L // 2

    def group_body(g, carry):
        base = g * GP
        pv = spans_v[pl.ds(g * L, L)]
        for k in range(GP):
            s0 = pv[2 * k]
            width = plsc.bitcast((pv[2 * k + 1] - s0).reshape(1), jnp.uint32)[0]
            for j in range(TW // L):
                off = plsc.bitcast(lanes_j[j] - s0, jnp.uint32)
                a_v[base + k, pl.ds(j * L, L)] = jnp.where(off <= width, one, zero)
        pltpu.async_copy(a_v.at[pl.ds(base, GP), :],
                         a_hbm.at[pl.ds(r0 + base, GP), :], sem)
        return carry

    lax.fori_loop(0, SPW // GP, group_body, 0)
    for g in range(SPW // GP):
        pltpu.make_async_copy(a_v.at[pl.ds(g * GP, GP), :],
                              a_hbm.at[pl.ds(r0 + g * GP, GP), :], sem).wait()


BB = 4


def _tc_reduce_body(mask_ref, expz_ref, seq_ref, out_ref):
    for q in range(BB):
        a = mask_ref[q] * expz_ref[q]
        att = a / jnp.sum(a, axis=1, keepdims=True)
        out_ref[q] = jnp.dot(att, seq_ref[q], preferred_element_type=jnp.float32)


def _tc_reduce(mask, expz, sequence_tensor):
    return pl.pallas_call(
        _tc_reduce_body,
        grid=(B // BB,),
        in_specs=[
            pl.BlockSpec((BB, S, TW), lambda i: (i, 0, 0)),
            pl.BlockSpec((BB, 1, TW), lambda i: (i, 0, 0)),
            pl.BlockSpec((BB, TW, D), lambda i: (i, 0, 0)),
        ],
        out_specs=pl.BlockSpec((BB, S, D), lambda i: (i, 0, 0)),
        out_shape=jax.ShapeDtypeStruct((B, S, D), jnp.float32),
    )(mask, expz, sequence_tensor)


def kernel(sequence_tensor, span_indices, w, b):
    expz = _tc_logits(sequence_tensor, w, b)
    mask = _sc_build_mask(span_indices.reshape(B * S * 2))
    return _tc_reduce(mask.reshape(B, S, TW), expz, sequence_tensor)


# speedup vs baseline: 1.0426x; 1.0426x over previous
import functools

import jax
import jax.numpy as jnp
from jax import lax
from jax.experimental import pallas as pl
from jax.experimental.pallas import tpu as pltpu
from jax.experimental.pallas import tpu_sc as plsc

B, T, D, S = 8, 2048, 512, 256
TW = 256
L = 16
NW = 32
SPW = (B * S) // NW


HB = 4


def _tc_logits_body(seq_ref, w_ref, b_ref, expz_ref):
    seq = seq_ref[...].reshape(HB * TW, D)
    z = jnp.dot(seq, w_ref[...], preferred_element_type=jnp.float32)
    z = z.reshape(HB, TW) + b_ref[0]
    mx = jnp.max(z, axis=1, keepdims=True)
    expz_ref[...] = jnp.exp(z - mx).reshape(HB, 1, TW)


def _tc_logits(sequence_tensor, w, b):
    return pl.pallas_call(
        _tc_logits_body,
        grid=(B // HB,),
        in_specs=[
            pl.BlockSpec((HB, TW, D), lambda i: (i, 0, 0)),
            pl.BlockSpec((D, 1), lambda i: (0, 0)),
            pl.BlockSpec((1,), lambda i: (0,)),
        ],
        out_specs=pl.BlockSpec((HB, 1, TW), lambda i: (i, 0, 0)),
        out_shape=jax.ShapeDtypeStruct((B, 1, TW), jnp.float32),
    )(sequence_tensor, w, b)


_sc_mesh = plsc.VectorSubcoreMesh(core_axis_name="c", subcore_axis_name="s")


@functools.partial(
    pl.kernel,
    mesh=_sc_mesh,
    out_type=jax.ShapeDtypeStruct((B * S, TW), jnp.float32),
    scratch_types=[
        pltpu.VMEM((SPW,), jnp.int32),
        pltpu.VMEM((SPW,), jnp.int32),
        pltpu.VMEM((SPW, TW), jnp.float32),
        pltpu.SemaphoreType.DMA,
    ],
)
def _sc_build_mask(starts_hbm, ends_hbm, a_hbm, starts_v, ends_v, a_v, sem):
    wid = lax.axis_index("s") * 2 + lax.axis_index("c")
    r0 = wid * SPW
    pltpu.sync_copy(starts_hbm.at[pl.ds(r0, SPW)], starts_v)
    pltpu.sync_copy(ends_hbm.at[pl.ds(r0, SPW)], ends_v)

    iota = lax.iota(jnp.int32, L)
    lanes_j = [iota + j * L for j in range(TW // L)]
    one = jnp.full((L,), 1.0, dtype=jnp.float32)
    zero = jnp.zeros((L,), dtype=jnp.float32)

    def group_body(g, carry):
        base = g * L
        sv = starts_v[pl.ds(base, L)]
        ev = ends_v[pl.ds(base, L)]
        for k in range(L):
            s0 = sv[k]
            width = plsc.bitcast((ev[k] - s0).reshape(1), jnp.uint32)[0]
            for j in range(TW // L):
                off = plsc.bitcast(lanes_j[j] - s0, jnp.uint32)
                a_v[base + k, pl.ds(j * L, L)] = jnp.where(off <= width, one, zero)
        pltpu.async_copy(a_v.at[pl.ds(base, L), :],
                         a_hbm.at[pl.ds(r0 + base, L), :], sem)
        return carry

    lax.fori_loop(0, SPW // L, group_body, 0)
    for g in range(SPW // L):
        pltpu.make_async_copy(a_v.at[pl.ds(g * L, L), :],
                              a_hbm.at[pl.ds(r0 + g * L, L), :], sem).wait()


BB = 4


def _tc_reduce_body(mask_ref, expz_ref, seq_ref, out_ref):
    for q in range(BB):
        a = mask_ref[q] * expz_ref[q]
        att = a / jnp.sum(a, axis=1, keepdims=True)
        out_ref[q] = jnp.dot(att, seq_ref[q], preferred_element_type=jnp.float32)


def _tc_reduce(mask, expz, sequence_tensor):
    return pl.pallas_call(
        _tc_reduce_body,
        grid=(B // BB,),
        in_specs=[
            pl.BlockSpec((BB, S, TW), lambda i: (i, 0, 0)),
            pl.BlockSpec((BB, 1, TW), lambda i: (i, 0, 0)),
            pl.BlockSpec((BB, TW, D), lambda i: (i, 0, 0)),
        ],
        out_specs=pl.BlockSpec((BB, S, D), lambda i: (i, 0, 0)),
        out_shape=jax.ShapeDtypeStruct((B, S, D), jnp.float32),
    )(mask, expz, sequence_tensor)


def kernel(sequence_tensor, span_indices, w, b):
    expz = _tc_logits(sequence_tensor, w, b)
    starts = span_indices[:, :, 0].reshape(B * S)
    ends = span_indices[:, :, 1].reshape(B * S)
    mask = _sc_build_mask(starts, ends)
    return _tc_reduce(mask.reshape(B, S, TW), expz, sequence_tensor)
